# Initial kernel scaffold; baseline (speedup 1.0000x reference)
import jax
import jax.numpy as jnp
from jax.experimental import pallas as pl


def kernel(latents, embedding_weight):
    raise NotImplementedError("write your pallas kernel here")


# speedup vs baseline: 1.3140x; 1.3140x over previous
import functools

import jax
import jax.numpy as jnp
from jax.experimental import pallas as pl
from jax.experimental.pallas import tpu as pltpu
from jax.experimental.pallas import tpu_sc as plsc

K = 8192
D = 256
N = 8192
BETA = 0.25

TILE_N = 256
GATHER_WINDOW = 64


def _dist_argmin_kernel(xb_ref, ebt_ref, a_ref, b_ref, idx_ref, lsum_ref):
    mm = jax.lax.dot_general(
        xb_ref[...], ebt_ref[...], (((1,), (0,)), ((), ())),
        preferred_element_type=jnp.float32)
    d = (a_ref[...] + b_ref[...]) - 2.0 * mm
    m = jnp.min(d, axis=1, keepdims=True)
    cols = jax.lax.broadcasted_iota(jnp.int32, d.shape, 1)
    idx = jnp.min(jnp.where(d == m, cols, K), axis=1, keepdims=True)
    idx_ref[...] = idx

    @pl.when(pl.program_id(0) == 0)
    def _():
        lsum_ref[...] = jnp.zeros_like(lsum_ref)

    lsum_ref[...] = lsum_ref[...] + jnp.sum(m).reshape(1, 1)


def _tc_dist_argmin(xb, ebt, a, b):
    return pl.pallas_call(
        _dist_argmin_kernel,
        grid=(N // TILE_N,),
        in_specs=[
            pl.BlockSpec((TILE_N, D), lambda i: (i, 0)),
            pl.BlockSpec((D, K), lambda i: (0, 0)),
            pl.BlockSpec((TILE_N, 1), lambda i: (i, 0)),
            pl.BlockSpec((1, K), lambda i: (0, 0)),
        ],
        out_specs=[
            pl.BlockSpec((TILE_N, 1), lambda i: (i, 0)),
            pl.BlockSpec((1, 1), lambda i: (0, 0)),
        ],
        out_shape=[
            jax.ShapeDtypeStruct((N, 1), jnp.int32),
            jax.ShapeDtypeStruct((1, 1), jnp.float32),
        ],
    )(xb, ebt, a, b)


_SC_CORES = 2
_SC_SUBCORES = 16
_SC_WORKERS = _SC_CORES * _SC_SUBCORES
_ROWS_PER_WORKER = N // _SC_WORKERS


def _sc_gather(qdata, idx1d):
    mesh = plsc.VectorSubcoreMesh(
        core_axis_name="c", subcore_axis_name="s",
        num_cores=_SC_CORES, num_subcores=_SC_SUBCORES)

    @functools.partial(
        pl.kernel,
        out_type=jax.ShapeDtypeStruct((N, D), jnp.float32),
        mesh=mesh,
        scratch_types=[
            pltpu.VMEM((_ROWS_PER_WORKER,), jnp.int32),
            pltpu.VMEM((_ROWS_PER_WORKER, D), jnp.float32),
            pltpu.SemaphoreType.DMA,
        ],
    )
    def gather_kernel(table_hbm, idx_hbm, out_hbm, idx_v, rows_v, sem):
        wid = jax.lax.axis_index("s") * _SC_CORES + jax.lax.axis_index("c")
        base = wid * _ROWS_PER_WORKER
        pltpu.sync_copy(idx_hbm.at[pl.ds(base, _ROWS_PER_WORKER)], idx_v)
        pltpu.async_copy(table_hbm.at[idx_v], rows_v, sem).wait()
        pltpu.sync_copy(rows_v, out_hbm.at[pl.ds(base, _ROWS_PER_WORKER)])

    return gather_kernel(qdata, idx1d)


def kernel(latents, embedding_weight):
    x = jnp.transpose(latents, (0, 2, 3, 1))
    lat_shape = x.shape
    flat = x.reshape(-1, D)

    a = jnp.sum(flat ** 2, axis=1, keepdims=True)
    b = jnp.sum(embedding_weight ** 2, axis=1)[None, :]
    xb = flat.astype(jnp.bfloat16)
    ebt = embedding_weight.astype(jnp.bfloat16).T
    qdata = embedding_weight.astype(jnp.bfloat16).astype(jnp.float32)

    idx, lsum = _tc_dist_argmin(xb, ebt, a, b)

    q = _sc_gather(qdata, idx.reshape(N))

    quantized_st = flat + (q - flat)
    out = jnp.transpose(quantized_st.reshape(lat_shape), (0, 3, 1, 2))
    loss = (lsum / jnp.float32(N * D)).reshape(())
    return (out, loss, BETA * loss)
